# baseline (speedup 1.0000x reference)
import functools

import jax
import jax.numpy as jnp
from jax import lax
from jax.experimental import pallas as pl
from jax.experimental.pallas import tpu as pltpu
from jax.experimental.pallas import tpu_sc as plsc

_NC = 2
_NS = 16
_NW = _NC * _NS
_G = 128
_D = 64
_LANES = 16

_V = 1000000
_TC = 2048
_SPLIT = 244 * _TC
_T2ROWS = _V - _SPLIT + _SPLIT
_TBLOCKS = 245


def _tc_transpose_block(x_ref, x2_ref, o_ref):
    o_ref[...] = jnp.concatenate([x_ref[...].T, x2_ref[...].T], axis=1)


def _build_tc_transpose():
    return pl.pallas_call(
        _tc_transpose_block,
        grid=(_TBLOCKS,),
        in_specs=[pl.BlockSpec((_D, _TC), lambda i: (0, i)),
                  pl.BlockSpec((_D, _TC), lambda i: (0, i + 244))],
        out_specs=pl.BlockSpec((_TC, 2 * _D), lambda i: (i, 0)),
        out_shape=jax.ShapeDtypeStruct((_V - _SPLIT + _SPLIT + 576, 2 * _D),
                                       jnp.float32),
    )


def _build_gather(num_granules, total_tokens):
    mesh = plsc.VectorSubcoreMesh(core_axis_name="c", subcore_axis_name="s")

    @functools.partial(
        pl.kernel,
        mesh=mesh,
        compiler_params=pltpu.CompilerParams(needs_layout_passes=False),
        out_type=jax.ShapeDtypeStruct((total_tokens, _D), jnp.float32),
        scratch_types=[
            pltpu.VMEM((num_granules, _G), jnp.int32),
            pltpu.VMEM((2, _G), jnp.int32),
            pltpu.VMEM((2, _G, 128), jnp.float32),
            pltpu.VMEM((2, _G // 8, 8, _D), jnp.float32),
            pltpu.SemaphoreType.DMA((2,)),
            pltpu.SemaphoreType.DMA((2,)),
        ],
    )
    def body(ids_hbm, t2_hbm, out_hbm, idx_v, pair_v, rows_v, sel_v,
             gsem, wsem):
        wid = lax.axis_index("s") * _NC + lax.axis_index("c")
        pltpu.sync_copy(ids_hbm.at[wid], idx_v)
        out3 = out_hbm.reshape(total_tokens // 8, 8, _D)

        def start_gather(g, p):
            def mk_pair(j, c2):
                ids16 = idx_v[g, pl.ds(j * _LANES, _LANES)]
                pair_v[p, pl.ds(j * _LANES, _LANES)] = jnp.where(
                    ids16 >= _SPLIT, ids16 - _SPLIT, ids16)
                return c2
            lax.fori_loop(0, _G // _LANES, mk_pair, 0)
            pltpu.async_copy(t2_hbm.at[pair_v.at[p]], rows_v.at[p],
                             gsem.at[p])

        def wait_gather(g, p):
            pltpu.make_async_copy(t2_hbm.at[pair_v.at[p]], rows_v.at[p],
                                  gsem.at[p]).wait()

        def wait_write(p):
            pltpu.make_async_copy(sel_v.at[p], out3.at[pl.ds(0, _G // 8)],
                                  wsem.at[p]).wait()

        start_gather(0, 0)

        def step(g, carry):
            p = g & 1
            @pl.when(g + 1 < num_granules)
            def _():
                start_gather(g + 1, 1 - p)
            wait_gather(g, p)
            @pl.when(g >= 2)
            def _():
                wait_write(p)

            for j in range(_G // _LANES):
                ids16 = idx_v[g, pl.ds(j * _LANES, _LANES)]
                for k in range(_LANES):
                    col = jnp.where(ids16[k] >= _SPLIT, _D, 0)
                    r = j * _LANES + k
                    for c in range(4):
                        vals = rows_v[p, r, pl.ds(col + c * _LANES, _LANES)]
                        sel_v[p, j * 2 + k // 8, k % 8,
                              pl.ds(c * _LANES, _LANES)] = vals

            base8 = (wid * num_granules + g) * (_G // 8)
            pltpu.async_copy(sel_v.at[p], out3.at[pl.ds(base8, _G // 8)],
                             wsem.at[p])
            return carry

        lax.fori_loop(0, num_granules, step, 0)
        wait_write(num_granules & 1)
        wait_write(1 - (num_granules & 1))

    return body


def kernel(token_ids, weight):
    batch, seq = token_ids.shape
    vocab, dim = weight.shape
    total = batch * seq
    num_granules = total // (_NW * _G)

    wt = weight.T
    t2 = _build_tc_transpose()(wt, wt)
    ids = token_ids.reshape(_NW, num_granules, _G).astype(jnp.int32)
    out = _build_gather(num_granules, total)(ids, t2)
    return out.reshape(batch, seq, dim)


# speedup vs baseline: 1.0796x; 1.0796x over previous
import functools

import jax
import jax.numpy as jnp
from jax import lax
from jax.experimental import pallas as pl
from jax.experimental.pallas import tpu as pltpu
from jax.experimental.pallas import tpu_sc as plsc

_NC = 2
_NS = 16
_NW = _NC * _NS
_G = 128
_D = 64
_LANES = 16

_V = 1000000
_TC = 4096
_SPLIT = 122 * _TC
_T2ROWS = _V - _SPLIT + _SPLIT
_TBLOCKS = 123


def _tc_transpose_block(x_ref, x2_ref, o_ref):
    o_ref[...] = jnp.concatenate([x_ref[...].T, x2_ref[...].T], axis=1)


def _build_tc_transpose():
    return pl.pallas_call(
        _tc_transpose_block,
        grid=(_TBLOCKS,),
        in_specs=[pl.BlockSpec((_D, _TC), lambda i: (0, i)),
                  pl.BlockSpec((_D, _TC), lambda i: (0, i + 122))],
        out_specs=pl.BlockSpec((_TC, 2 * _D), lambda i: (i, 0)),
        out_shape=jax.ShapeDtypeStruct((_V - _SPLIT + _SPLIT + 576, 2 * _D),
                                       jnp.float32),
    )


def _build_gather(num_granules, total_tokens):
    mesh = plsc.VectorSubcoreMesh(core_axis_name="c", subcore_axis_name="s")

    @functools.partial(
        pl.kernel,
        mesh=mesh,
        compiler_params=pltpu.CompilerParams(needs_layout_passes=False),
        out_type=jax.ShapeDtypeStruct((total_tokens, _D), jnp.float32),
        scratch_types=[
            pltpu.VMEM((num_granules, _G), jnp.int32),
            pltpu.VMEM((2, _G), jnp.int32),
            pltpu.VMEM((2, _G, 128), jnp.float32),
            pltpu.VMEM((2, _G // 8, 8, _D), jnp.float32),
            pltpu.SemaphoreType.DMA((2,)),
            pltpu.SemaphoreType.DMA((2,)),
        ],
    )
    def body(ids_hbm, t2_hbm, out_hbm, idx_v, pair_v, rows_v, sel_v,
             gsem, wsem):
        wid = lax.axis_index("s") * _NC + lax.axis_index("c")
        pltpu.sync_copy(ids_hbm.at[wid], idx_v)
        out3 = out_hbm.reshape(total_tokens // 8, 8, _D)

        def start_gather(g, p):
            def mk_pair(j, c2):
                ids16 = idx_v[g, pl.ds(j * _LANES, _LANES)]
                pair_v[p, pl.ds(j * _LANES, _LANES)] = jnp.where(
                    ids16 >= _SPLIT, ids16 - _SPLIT, ids16)
                return c2
            lax.fori_loop(0, _G // _LANES, mk_pair, 0)
            pltpu.async_copy(t2_hbm.at[pair_v.at[p]], rows_v.at[p],
                             gsem.at[p])

        def wait_gather(g, p):
            pltpu.make_async_copy(t2_hbm.at[pair_v.at[p]], rows_v.at[p],
                                  gsem.at[p]).wait()

        def wait_write(p):
            pltpu.make_async_copy(sel_v.at[p], out3.at[pl.ds(0, _G // 8)],
                                  wsem.at[p]).wait()

        start_gather(0, 0)

        def step(g, carry):
            p = g & 1
            @pl.when(g + 1 < num_granules)
            def _():
                start_gather(g + 1, 1 - p)
            wait_gather(g, p)
            @pl.when(g >= 2)
            def _():
                wait_write(p)

            for j in range(_G // _LANES):
                ids16 = idx_v[g, pl.ds(j * _LANES, _LANES)]
                for k in range(_LANES):
                    col = jnp.where(ids16[k] >= _SPLIT, _D, 0)
                    r = j * _LANES + k
                    for c in range(4):
                        vals = rows_v[p, r, pl.ds(col + c * _LANES, _LANES)]
                        sel_v[p, j * 2 + k // 8, k % 8,
                              pl.ds(c * _LANES, _LANES)] = vals

            base8 = (wid * num_granules + g) * (_G // 8)
            pltpu.async_copy(sel_v.at[p], out3.at[pl.ds(base8, _G // 8)],
                             wsem.at[p])
            return carry

        lax.fori_loop(0, num_granules, step, 0)
        wait_write(num_granules & 1)
        wait_write(1 - (num_granules & 1))

    return body


def kernel(token_ids, weight):
    batch, seq = token_ids.shape
    vocab, dim = weight.shape
    total = batch * seq
    num_granules = total // (_NW * _G)

    wt = weight.T
    t2 = _build_tc_transpose()(wt, wt)
    ids = token_ids.reshape(_NW, num_granules, _G).astype(jnp.int32)
    out = _build_gather(num_granules, total)(ids, t2)
    return out.reshape(batch, seq, dim)


# speedup vs baseline: 1.1216x; 1.0390x over previous
import functools

import jax
import jax.numpy as jnp
from jax import lax
from jax.experimental import pallas as pl
from jax.experimental.pallas import tpu as pltpu
from jax.experimental.pallas import tpu_sc as plsc

_NC = 2
_NS = 16
_NW = _NC * _NS
_G = 128
_D = 64
_LANES = 16

_V = 1000000
_TC = 8192
_SPLIT = 61 * _TC
_T2ROWS = _V - _SPLIT + _SPLIT
_TBLOCKS = 62


def _tc_transpose_block(x_ref, x2_ref, o_ref):
    o_ref[...] = jnp.concatenate([x_ref[...].T, x2_ref[...].T], axis=1)


def _build_tc_transpose():
    return pl.pallas_call(
        _tc_transpose_block,
        grid=(_TBLOCKS,),
        in_specs=[pl.BlockSpec((_D, _TC), lambda i: (0, i)),
                  pl.BlockSpec((_D, _TC), lambda i: (0, i + 61))],
        out_specs=pl.BlockSpec((_TC, 2 * _D), lambda i: (i, 0)),
        out_shape=jax.ShapeDtypeStruct((_V - _SPLIT + _SPLIT + 576, 2 * _D),
                                       jnp.float32),
    )


def _build_gather(num_granules, total_tokens):
    mesh = plsc.VectorSubcoreMesh(core_axis_name="c", subcore_axis_name="s")

    @functools.partial(
        pl.kernel,
        mesh=mesh,
        compiler_params=pltpu.CompilerParams(needs_layout_passes=False),
        out_type=jax.ShapeDtypeStruct((total_tokens, _D), jnp.float32),
        scratch_types=[
            pltpu.VMEM((num_granules, _G), jnp.int32),
            pltpu.VMEM((2, _G), jnp.int32),
            pltpu.VMEM((2, _G, 128), jnp.float32),
            pltpu.VMEM((2, _G // 8, 8, _D), jnp.float32),
            pltpu.SemaphoreType.DMA((2,)),
            pltpu.SemaphoreType.DMA((2,)),
        ],
    )
    def body(ids_hbm, t2_hbm, out_hbm, idx_v, pair_v, rows_v, sel_v,
             gsem, wsem):
        wid = lax.axis_index("s") * _NC + lax.axis_index("c")
        pltpu.sync_copy(ids_hbm.at[wid], idx_v)
        out3 = out_hbm.reshape(total_tokens // 8, 8, _D)

        def start_gather(g, p):
            def mk_pair(j, c2):
                ids16 = idx_v[g, pl.ds(j * _LANES, _LANES)]
                pair_v[p, pl.ds(j * _LANES, _LANES)] = jnp.where(
                    ids16 >= _SPLIT, ids16 - _SPLIT, ids16)
                return c2
            lax.fori_loop(0, _G // _LANES, mk_pair, 0)
            pltpu.async_copy(t2_hbm.at[pair_v.at[p]], rows_v.at[p],
                             gsem.at[p])

        def wait_gather(g, p):
            pltpu.make_async_copy(t2_hbm.at[pair_v.at[p]], rows_v.at[p],
                                  gsem.at[p]).wait()

        def wait_write(p):
            pltpu.make_async_copy(sel_v.at[p], out3.at[pl.ds(0, _G // 8)],
                                  wsem.at[p]).wait()

        start_gather(0, 0)

        def step(g, carry):
            p = g & 1
            @pl.when(g + 1 < num_granules)
            def _():
                start_gather(g + 1, 1 - p)
            wait_gather(g, p)
            @pl.when(g >= 2)
            def _():
                wait_write(p)

            for j in range(_G // _LANES):
                ids16 = idx_v[g, pl.ds(j * _LANES, _LANES)]
                for k in range(_LANES):
                    col = jnp.where(ids16[k] >= _SPLIT, _D, 0)
                    r = j * _LANES + k
                    for c in range(4):
                        vals = rows_v[p, r, pl.ds(col + c * _LANES, _LANES)]
                        sel_v[p, j * 2 + k // 8, k % 8,
                              pl.ds(c * _LANES, _LANES)] = vals

            base8 = (wid * num_granules + g) * (_G // 8)
            pltpu.async_copy(sel_v.at[p], out3.at[pl.ds(base8, _G // 8)],
                             wsem.at[p])
            return carry

        lax.fori_loop(0, num_granules, step, 0)
        wait_write(num_granules & 1)
        wait_write(1 - (num_granules & 1))

    return body


def kernel(token_ids, weight):
    batch, seq = token_ids.shape
    vocab, dim = weight.shape
    total = batch * seq
    num_granules = total // (_NW * _G)

    wt = weight.T
    t2 = _build_tc_transpose()(wt, wt)
    ids = token_ids.reshape(_NW, num_granules, _G).astype(jnp.int32)
    out = _build_gather(num_granules, total)(ids, t2)
    return out.reshape(batch, seq, dim)


# speedup vs baseline: 1.1244x; 1.0024x over previous
import functools

import jax
import jax.numpy as jnp
from jax import lax
from jax.experimental import pallas as pl
from jax.experimental.pallas import tpu as pltpu
from jax.experimental.pallas import tpu_sc as plsc

_NC = 2
_NS = 16
_NW = _NC * _NS
_G = 128
_D = 64
_LANES = 16

_V = 1000000
_TC = 8192
_SPLIT = 61 * _TC
_T2ROWS = _V - _SPLIT + _SPLIT
_TBLOCKS = 62


def _tc_transpose_block(x_ref, x2_ref, o_ref):
    o_ref[...] = jnp.concatenate([x_ref[...].T, x2_ref[...].T], axis=1)


def _build_tc_transpose():
    return pl.pallas_call(
        _tc_transpose_block,
        grid=(_TBLOCKS,),
        in_specs=[pl.BlockSpec((_D, _TC), lambda i: (0, i)),
                  pl.BlockSpec((_D, _TC), lambda i: (0, i + 61))],
        out_specs=pl.BlockSpec((_TC, 2 * _D), lambda i: (i, 0)),
        out_shape=jax.ShapeDtypeStruct((_V - _SPLIT + _SPLIT + 576, 2 * _D),
                                       jnp.float32),
    )


def _build_gather(num_granules, total_tokens):
    mesh = plsc.VectorSubcoreMesh(core_axis_name="c", subcore_axis_name="s")

    @functools.partial(
        pl.kernel,
        mesh=mesh,
        compiler_params=pltpu.CompilerParams(needs_layout_passes=False),
        out_type=jax.ShapeDtypeStruct((total_tokens, _D), jnp.float32),
        scratch_types=[
            pltpu.VMEM((num_granules, _G), jnp.int32),
            pltpu.VMEM((2, _G), jnp.int32),
            pltpu.VMEM((2, _G, 128), jnp.float32),
            pltpu.VMEM((2, _G // 8, 8, _D), jnp.float32),
            pltpu.SemaphoreType.DMA((2,)),
            pltpu.SemaphoreType.DMA((2,)),
        ],
    )
    def body(ids_hbm, t2_hbm, out_hbm, idx_v, pair_v, rows_v, sel_v,
             gsem, wsem):
        wid = lax.axis_index("s") * _NC + lax.axis_index("c")
        pltpu.sync_copy(ids_hbm.at[wid], idx_v)
        out3 = out_hbm.reshape(total_tokens // 8, 8, _D)

        def start_gather(g, p):
            def mk_pair(j, c2):
                ids16 = idx_v[g, pl.ds(j * _LANES, _LANES)]
                pair_v[p, pl.ds(j * _LANES, _LANES)] = jnp.where(
                    ids16 >= _SPLIT, ids16 - _SPLIT, ids16)
                return c2
            lax.fori_loop(0, _G // _LANES, mk_pair, 0)
            pltpu.async_copy(t2_hbm.at[pair_v.at[p]], rows_v.at[p],
                             gsem.at[p])

        def wait_gather(g, p):
            pltpu.make_async_copy(t2_hbm.at[pair_v.at[p]], rows_v.at[p],
                                  gsem.at[p]).wait()

        def wait_write(p):
            pltpu.make_async_copy(sel_v.at[p], out3.at[pl.ds(0, _G // 8)],
                                  wsem.at[p]).wait()

        start_gather(0, 0)

        def step(g, carry):
            p = g & 1
            @pl.when(g + 1 < num_granules)
            def _():
                start_gather(g + 1, 1 - p)
            wait_gather(g, p)
            @pl.when(g >= 2)
            def _():
                wait_write(p)

            for j in range(_G // _LANES):
                ids16 = idx_v[g, pl.ds(j * _LANES, _LANES)]
                cols = jnp.where(ids16 >= _SPLIT, _D, 0)
                for k in range(_LANES):
                    col = cols[k]
                    r = j * _LANES + k
                    for c in range(4):
                        vals = rows_v[p, r, pl.ds(col + c * _LANES, _LANES)]
                        sel_v[p, j * 2 + k // 8, k % 8,
                              pl.ds(c * _LANES, _LANES)] = vals

            base8 = (wid * num_granules + g) * (_G // 8)
            pltpu.async_copy(sel_v.at[p], out3.at[pl.ds(base8, _G // 8)],
                             wsem.at[p])
            return carry

        lax.fori_loop(0, num_granules, step, 0)
        wait_write(num_granules & 1)
        wait_write(1 - (num_granules & 1))

    return body


def kernel(token_ids, weight):
    batch, seq = token_ids.shape
    vocab, dim = weight.shape
    total = batch * seq
    num_granules = total // (_NW * _G)

    wt = weight.T
    t2 = _build_tc_transpose()(wt, wt)
    ids = token_ids.reshape(_NW, num_granules, _G).astype(jnp.int32)
    out = _build_gather(num_granules, total)(ids, t2)
    return out.reshape(batch, seq, dim)
